# baseline (speedup 1.0000x reference)
import functools

import jax
import jax.numpy as jnp
from jax import lax
from jax.experimental import pallas as pl
from jax.experimental.pallas import tpu as pltpu
from jax.experimental.pallas import tpu_sc as plsc

NUM_TEAMS = 100000
EMBED_DIM = 128
BATCH = 16384
FEAT2 = 2 * EMBED_DIM
N_WORKERS = 32
ROWS_PER_W = BATCH // N_WORKERS
CHUNK = 128
N_CHUNKS = ROWS_PER_W // CHUNK
TILE = 512
N_TILES = BATCH // TILE
EPS = 1e-5


def _sc_gather_body(table, idx1, idx2, feats, idx_v, rows_v, sem):
    wid = lax.axis_index("s") * 2 + lax.axis_index("c")
    base = wid * ROWS_PER_W
    irow = wid * N_CHUNKS
    for half, idx_hbm in ((0, idx1), (1, idx2)):
        pltpu.sync_copy(idx_hbm.at[pl.ds(irow, N_CHUNKS)], idx_v)
        copies = [
            pltpu.async_copy(
                table.at[idx_v.at[j]],
                rows_v.at[pl.ds(j * CHUNK, CHUNK)],
                sem,
            )
            for j in range(N_CHUNKS)
        ]
        for cp in copies:
            cp.wait()
        pltpu.sync_copy(
            rows_v,
            feats.at[pl.ds(base, ROWS_PER_W), pl.ds(half * EMBED_DIM, EMBED_DIM)],
        )


_sc_gather = functools.partial(
    pl.kernel,
    mesh=plsc.VectorSubcoreMesh(core_axis_name="c", subcore_axis_name="s"),
    out_type=jax.ShapeDtypeStruct((BATCH, FEAT2), jnp.float32),
    scratch_types=[
        pltpu.VMEM((N_CHUNKS, CHUNK), jnp.int32),
        pltpu.VMEM((ROWS_PER_W, EMBED_DIM), jnp.float32),
        pltpu.SemaphoreType.DMA,
    ],
)(_sc_gather_body)


def _tc_bn_body(feats_ref, score_ref, gw_ref, scal_ref, out_ref, acc_ref, ss_ref):
    p = pl.program_id(0)
    t = pl.program_id(1)

    @pl.when(jnp.logical_and(p == 0, t == 0))
    def _init():
        acc_ref[...] = jnp.zeros_like(acc_ref)
        ss_ref[0] = 0.0
        ss_ref[1] = 0.0

    @pl.when(p == 0)
    def _stats():
        x = feats_ref[...]
        acc_ref[0:1, :] = acc_ref[0:1, :] + jnp.sum(x, axis=0, keepdims=True)
        acc_ref[1:2, :] = acc_ref[1:2, :] + jnp.sum(x * x, axis=0, keepdims=True)
        s = score_ref[...]
        ss_ref[0] = ss_ref[0] + jnp.sum(s)
        ss_ref[1] = ss_ref[1] + jnp.sum(s * s)

    @pl.when(p == 1)
    def _apply():
        inv_n = 1.0 / BATCH
        mean = acc_ref[0:1, :] * inv_n
        var = acc_ref[1:2, :] * inv_n - mean * mean
        c = gw_ref[...] * lax.rsqrt(var + EPS)
        smean = ss_ref[0] * inv_n
        svar = ss_ref[1] * inv_n - smean * smean
        cs = scal_ref[0, 0] * lax.rsqrt(svar + EPS)
        const = scal_ref[0, 1] - jnp.sum(c * mean) - cs * smean
        x = feats_ref[...]
        z = jnp.sum(x * c, axis=1, keepdims=True)
        z = z + score_ref[...] * cs + const
        out_ref[...] = jax.nn.sigmoid(z)


def _tc_bn(feats, score, gw256, scal):
    return pl.pallas_call(
        _tc_bn_body,
        grid=(2, N_TILES),
        in_specs=[
            pl.BlockSpec((TILE, FEAT2), lambda p, t: (t, 0)),
            pl.BlockSpec((TILE, 1), lambda p, t: (t, 0)),
            pl.BlockSpec((1, FEAT2), lambda p, t: (0, 0)),
            pl.BlockSpec(memory_space=pltpu.SMEM),
        ],
        out_specs=pl.BlockSpec((TILE, 1), lambda p, t: (t, 0)),
        out_shape=jax.ShapeDtypeStruct((BATCH, 1), jnp.float32),
        scratch_shapes=[
            pltpu.VMEM((2, FEAT2), jnp.float32),
            pltpu.SMEM((2,), jnp.float32),
        ],
    )(feats, score, gw256, scal)


def kernel(idsTensor, table, gamma, beta, W, b):
    idx1 = idsTensor[:, 0].astype(jnp.int32).reshape(128, 128)
    idx2 = idsTensor[:, 1].astype(jnp.int32).reshape(128, 128)
    score = idsTensor[:, 2:3]
    w = W[0]
    gw = gamma * w
    gw256 = gw[:FEAT2].reshape(1, FEAT2)
    scal = jnp.stack([gw[FEAT2], b[0] + jnp.sum(beta * w)]).reshape(1, 2)
    feats = _sc_gather(table, idx1, idx2)
    return _tc_bn(feats, score, gw256, scal)


# speedup vs baseline: 1.9875x; 1.9875x over previous
import functools

import jax
import jax.numpy as jnp
from jax import lax
from jax.experimental import pallas as pl
from jax.experimental.pallas import tpu as pltpu
from jax.experimental.pallas import tpu_sc as plsc

NUM_TEAMS = 100000
EMBED_DIM = 128
BATCH = 16384
FEAT2 = 2 * EMBED_DIM
N_WORKERS = 32
ROWS_PER_W = BATCH // N_WORKERS
CHUNK = 128
N_CHUNKS = ROWS_PER_W // CHUNK
N_UNITS = 2 * N_CHUNKS
NBUF = 6
EPS = 1e-5

N_TC_CHUNKS = 4
CROWS = BATCH // N_TC_CHUNKS
CROWS128 = CROWS // 128


def _sc_gather_body(table, idx1, idx2, feats, idx_v, bufs, *sems):
    gsems = sems[:NBUF]
    wsems = sems[NBUF:]
    wid = lax.axis_index("s") * 2 + lax.axis_index("c")
    base = wid * ROWS_PER_W
    irow = wid * N_CHUNKS
    pltpu.sync_copy(idx1.at[pl.ds(irow, N_CHUNKS)], idx_v.at[pl.ds(0, N_CHUNKS)])
    pltpu.sync_copy(
        idx2.at[pl.ds(irow, N_CHUNKS)], idx_v.at[pl.ds(N_CHUNKS, N_CHUNKS)]
    )

    def buf_at(u):
        return bufs.at[pl.ds((u % NBUF) * CHUNK, CHUNK)]

    def feats_at(u):
        half, j = divmod(u, N_CHUNKS)
        return feats.at[
            pl.ds(base + j * CHUNK, CHUNK), pl.ds(half * EMBED_DIM, EMBED_DIM)
        ]

    gathers = {}
    writes = {}
    for u in range(min(NBUF, N_UNITS)):
        gathers[u] = pltpu.async_copy(table.at[idx_v.at[u]], buf_at(u), sems[u % NBUF])
    for u in range(N_UNITS):
        gathers[u].wait()
        writes[u] = pltpu.async_copy(buf_at(u), feats_at(u), wsems[u % NBUF])
        if u + NBUF < N_UNITS:
            writes[u].wait()
            gathers[u + NBUF] = pltpu.async_copy(
                table.at[idx_v.at[u + NBUF]], buf_at(u + NBUF), gsems[(u + NBUF) % NBUF]
            )
    for u in range(max(0, N_UNITS - NBUF), N_UNITS):
        writes[u].wait()


_sc_gather = functools.partial(
    pl.kernel,
    mesh=plsc.VectorSubcoreMesh(core_axis_name="c", subcore_axis_name="s"),
    out_type=jax.ShapeDtypeStruct((BATCH, FEAT2), jnp.float32),
    scratch_types=[
        pltpu.VMEM((N_UNITS, CHUNK), jnp.int32),
        pltpu.VMEM((NBUF * CHUNK, EMBED_DIM), jnp.float32),
    ]
    + [pltpu.SemaphoreType.DMA] * (2 * NBUF),
)(_sc_gather_body)


def _tc_bn_body(feats_hbm, s2d_ref, gw_ref, scal_ref, out_ref, x_ref, sems):
    copies = []
    for i in range(N_TC_CHUNKS):
        cp = pltpu.make_async_copy(
            feats_hbm.at[pl.ds(i * CROWS128, CROWS128)],
            x_ref.at[pl.ds(i * CROWS128, CROWS128)],
            sems.at[i],
        )
        cp.start()
        copies.append(cp)
    ssum = jnp.zeros((FEAT2,), jnp.float32)
    ssq = jnp.zeros((FEAT2,), jnp.float32)
    for i in range(N_TC_CHUNKS):
        copies[i].wait()
        x = x_ref[pl.ds(i * CROWS128, CROWS128)]
        ssum = ssum + jnp.sum(jnp.sum(x, axis=0), axis=0)
        ssq = ssq + jnp.sum(jnp.sum(x * x, axis=0), axis=0)
    s = s2d_ref[...]
    inv_n = 1.0 / BATCH
    smean = jnp.sum(s) * inv_n
    svar = jnp.sum(s * s) * inv_n - smean * smean
    mean = ssum * inv_n
    var = ssq * inv_n - mean * mean
    c = gw_ref[0, :] * lax.rsqrt(var + EPS)
    cs = scal_ref[0, 0] * lax.rsqrt(svar + EPS)
    const = scal_ref[0, 1] - jnp.sum(c * mean) - cs * smean
    for i in range(N_TC_CHUNKS):
        x = x_ref[pl.ds(i * CROWS128, CROWS128)]
        z = jnp.sum(x * c, axis=2)
        z = z + s2d_ref[pl.ds(i * CROWS128, CROWS128), :] * cs + const
        out_ref[pl.ds(i * CROWS128, CROWS128), :] = jax.nn.sigmoid(z)


def _tc_bn(feats3, s2d, gw256, scal):
    return pl.pallas_call(
        _tc_bn_body,
        in_specs=[
            pl.BlockSpec(memory_space=pltpu.MemorySpace.HBM),
            pl.BlockSpec(memory_space=pltpu.VMEM),
            pl.BlockSpec(memory_space=pltpu.VMEM),
            pl.BlockSpec(memory_space=pltpu.SMEM),
        ],
        out_specs=pl.BlockSpec(memory_space=pltpu.VMEM),
        out_shape=jax.ShapeDtypeStruct((128, 128), jnp.float32),
        scratch_shapes=[
            pltpu.VMEM((128, 128, FEAT2), jnp.float32),
            pltpu.SemaphoreType.DMA((N_TC_CHUNKS,)),
        ],
    )(feats3, s2d, gw256, scal)


def kernel(idsTensor, table, gamma, beta, W, b):
    idx1 = idsTensor[:, 0].astype(jnp.int32).reshape(128, 128)
    idx2 = idsTensor[:, 1].astype(jnp.int32).reshape(128, 128)
    s2d = idsTensor[:, 2].reshape(128, 128)
    w = W[0]
    gw = gamma * w
    gw256 = gw[:FEAT2].reshape(1, FEAT2)
    scal = jnp.stack([gw[FEAT2], b[0] + jnp.sum(beta * w)]).reshape(1, 2)
    feats = _sc_gather(table, idx1, idx2)
    feats3 = feats.reshape(128, 128, FEAT2)
    out = _tc_bn(feats3, s2d, gw256, scal)
    return out.reshape(BATCH, 1)


# speedup vs baseline: 2.2062x; 1.1100x over previous
import functools

import jax
import jax.numpy as jnp
from jax import lax
from jax.experimental import pallas as pl
from jax.experimental.pallas import tpu as pltpu
from jax.experimental.pallas import tpu_sc as plsc

NUM_TEAMS = 100000
EMBED_DIM = 128
BATCH = 16384
FEAT2 = 2 * EMBED_DIM
N_WORKERS = 32
ROWS_PER_W = BATCH // N_WORKERS
CHUNK = 128
N_CHUNKS = ROWS_PER_W // CHUNK
N_UNITS = 2 * N_CHUNKS
NBUF = 6
EPS = 1e-5

N_TC_CHUNKS = 4
CROWS = BATCH // N_TC_CHUNKS
CROWS128 = CROWS // 128


def _sc_gather_body(table, idx1, idx2, feats, idx_v, bufs, *sems):
    gsems = sems[:NBUF]
    wsems = sems[NBUF:]
    wid = lax.axis_index("s") * 2 + lax.axis_index("c")
    base = wid * ROWS_PER_W
    irow = wid * N_CHUNKS
    pltpu.sync_copy(idx1.at[pl.ds(irow, N_CHUNKS)], idx_v.at[pl.ds(0, N_CHUNKS)])
    pltpu.sync_copy(
        idx2.at[pl.ds(irow, N_CHUNKS)], idx_v.at[pl.ds(N_CHUNKS, N_CHUNKS)]
    )

    def buf_at(u):
        return bufs.at[pl.ds((u % NBUF) * CHUNK, CHUNK)]

    def feats_at(u):
        half, j = divmod(u, N_CHUNKS)
        return feats.at[
            pl.ds(base + j * CHUNK, CHUNK), pl.ds(half * EMBED_DIM, EMBED_DIM)
        ]

    gathers = {}
    writes = {}
    for u in range(min(NBUF, N_UNITS)):
        gathers[u] = pltpu.async_copy(table.at[idx_v.at[u]], buf_at(u), gsems[u % NBUF])
    for u in range(N_UNITS):
        gathers[u].wait()
        writes[u] = pltpu.async_copy(buf_at(u), feats_at(u), wsems[u % NBUF])
        if u + NBUF < N_UNITS:
            writes[u].wait()
            gathers[u + NBUF] = pltpu.async_copy(
                table.at[idx_v.at[u + NBUF]], buf_at(u + NBUF), gsems[(u + NBUF) % NBUF]
            )
    for u in range(max(0, N_UNITS - NBUF), N_UNITS):
        writes[u].wait()


_sc_gather = functools.partial(
    pl.kernel,
    mesh=plsc.VectorSubcoreMesh(core_axis_name="c", subcore_axis_name="s"),
    out_type=jax.ShapeDtypeStruct((BATCH, FEAT2), jnp.float32),
    scratch_types=[
        pltpu.VMEM((N_UNITS, CHUNK), jnp.int32),
        pltpu.VMEM((NBUF * CHUNK, EMBED_DIM), jnp.float32),
    ]
    + [pltpu.SemaphoreType.DMA] * (2 * NBUF),
)(_sc_gather_body)


def _tc_bn_body(feats_hbm, s2d_ref, gw_ref, scal_ref, out_ref, x_ref, z_ref, sems):
    copies = []
    for i in range(N_TC_CHUNKS):
        cp = pltpu.make_async_copy(
            feats_hbm.at[pl.ds(i * CROWS128, CROWS128)],
            x_ref.at[pl.ds(i * CROWS128, CROWS128)],
            sems.at[i],
        )
        cp.start()
        copies.append(cp)
    ssum = jnp.zeros((FEAT2,), jnp.float32)
    ssq = jnp.zeros((FEAT2,), jnp.float32)
    for i in range(N_TC_CHUNKS):
        copies[i].wait()
        x = x_ref[pl.ds(i * CROWS128, CROWS128)]
        ssum = ssum + jnp.sum(jnp.sum(x, axis=0), axis=0)
        ssq = ssq + jnp.sum(jnp.sum(x * x, axis=0), axis=0)
    s = s2d_ref[...]
    inv_n = 1.0 / BATCH
    smean = jnp.sum(s) * inv_n
    svar = jnp.sum(s * s) * inv_n - smean * smean
    mean = ssum * inv_n
    var = ssq * inv_n - mean * mean
    c = gw_ref[0, :] * lax.rsqrt(var + EPS)
    cs = scal_ref[0, 0] * lax.rsqrt(svar + EPS)
    const = scal_ref[0, 1] - jnp.sum(c * mean) - cs * smean
    for i in range(N_TC_CHUNKS):
        x = x_ref[pl.ds(i * CROWS128, CROWS128)]
        z_ref[pl.ds(i * CROWS128, CROWS128), :] = jnp.sum(x * c, axis=2)
    zz = z_ref[...] + s * cs + const
    out_ref[...] = jax.nn.sigmoid(zz)


def _tc_bn(feats3, s2d, gw256, scal):
    return pl.pallas_call(
        _tc_bn_body,
        in_specs=[
            pl.BlockSpec(memory_space=pltpu.MemorySpace.HBM),
            pl.BlockSpec(memory_space=pltpu.VMEM),
            pl.BlockSpec(memory_space=pltpu.VMEM),
            pl.BlockSpec(memory_space=pltpu.SMEM),
        ],
        out_specs=pl.BlockSpec(memory_space=pltpu.VMEM),
        out_shape=jax.ShapeDtypeStruct((128, 128), jnp.float32),
        scratch_shapes=[
            pltpu.VMEM((128, 128, FEAT2), jnp.float32),
            pltpu.VMEM((128, 128), jnp.float32),
            pltpu.SemaphoreType.DMA((N_TC_CHUNKS,)),
        ],
    )(feats3, s2d, gw256, scal)


def kernel(idsTensor, table, gamma, beta, W, b):
    idx1 = idsTensor[:, 0].astype(jnp.int32).reshape(128, 128)
    idx2 = idsTensor[:, 1].astype(jnp.int32).reshape(128, 128)
    s2d = idsTensor[:, 2].reshape(128, 128)
    w = W[0]
    gw = gamma * w
    gw256 = gw[:FEAT2].reshape(1, FEAT2)
    scal = jnp.stack([gw[FEAT2], b[0] + jnp.sum(beta * w)]).reshape(1, 2)
    feats = _sc_gather(table, idx1, idx2)
    feats3 = feats.reshape(128, 128, FEAT2)
    out = _tc_bn(feats3, s2d, gw256, scal)
    return out.reshape(BATCH, 1)
